# Initial kernel scaffold; baseline (speedup 1.0000x reference)
import jax
import jax.numpy as jnp
from jax.experimental import pallas as pl


def kernel(h, edge_index0, edge_index1, W0, al0, ar0, b0, W1, al1, ar1, b1, Ws1, bs1, Ws2):
    raise NotImplementedError("write your pallas kernel here")


# speedup vs baseline: 1.0313x; 1.0313x over previous
import functools

import jax
import jax.numpy as jnp
from jax.experimental import pallas as pl
from jax.experimental.pallas import tpu as pltpu

N = 10000
NP = 10240
IN = 128
H = 8
D = 64
HD = 512
E = 320000
HID = 128


def _dense_body(h_ref, w_ref, o_ref):
    o_ref[0] = jnp.dot(h_ref[...], w_ref[...], preferred_element_type=jnp.float32)


def _dense(hp, wext):
    nslice = wext.shape[1] // 128
    grid = (NP // 256, nslice)
    return pl.pallas_call(
        _dense_body,
        grid=grid,
        in_specs=[
            pl.BlockSpec((256, IN), lambda i, j: (i, 0)),
            pl.BlockSpec((IN, 128), lambda i, j: (0, j)),
        ],
        out_specs=pl.BlockSpec((1, 256, 128), lambda i, j: (j, i, 0)),
        out_shape=jax.ShapeDtypeStruct((nslice, NP, 128), jnp.float32),
    )(hp, wext)


def kernel(h, edge_index0, edge_index1, W0, al0, ar0, b0, W1, al1, ar1, b1, Ws1, bs1, Ws2):
    h = h.astype(jnp.float32)
    hp = jnp.pad(h, ((0, NP - N), (0, 0)))
    wcat = jnp.concatenate([W0, W1], axis=1)
    eye = jnp.eye(H, dtype=jnp.float32)
    blk0l = (al0[:, None, :] * eye[:, :, None]).transpose(1, 2, 0).reshape(HD, H)
    blk0r = (ar0[:, None, :] * eye[:, :, None]).transpose(1, 2, 0).reshape(HD, H)
    blk1l = (al1[:, None, :] * eye[:, :, None]).transpose(1, 2, 0).reshape(HD, H)
    blk1r = (ar1[:, None, :] * eye[:, :, None]).transpose(1, 2, 0).reshape(HD, H)
    z128 = jnp.zeros((HD, 128 - 4 * H), jnp.float32)
    B0 = jnp.concatenate([blk0l, blk0r, jnp.zeros((HD, 2 * H), jnp.float32), z128], axis=1)
    B1 = jnp.concatenate([jnp.zeros((HD, 2 * H), jnp.float32), blk1l, blk1r, z128], axis=1)
    Bm = jnp.concatenate([B0, jnp.zeros((HD, 128), jnp.float32)], axis=0) + \
         jnp.concatenate([jnp.zeros((HD, 128), jnp.float32), B1], axis=0)
    WB = wcat @ Bm
    wext = jnp.concatenate([wcat, WB], axis=1)

    fout = _dense(hp, wext)
    fcat = fout[:8].transpose(1, 0, 2).reshape(NP, 1024)[:N]
    EL = fout[8][:N]

    def gat(p, ei):
        src, dst = ei[0].astype(jnp.int32), ei[1].astype(jnp.int32)
        f = fcat[:, p * HD:(p + 1) * HD].reshape(N, H, D)
        el = EL[:, p * 16:p * 16 + 8]
        er = EL[:, p * 16 + 8:p * 16 + 16]
        e = jax.nn.leaky_relu(el[src] + er[dst], negative_slope=0.2)
        a = jnp.exp(e)
        s = jax.ops.segment_sum(a, dst, num_segments=N)
        alpha = a / (s[dst] + 1e-9)
        return jax.ops.segment_sum(f[src] * alpha[:, :, None], dst, num_segments=N)

    z0 = gat(0, edge_index0).reshape(N, HD)
    z1 = gat(1, edge_index1).reshape(N, HD)
    zb0 = jax.nn.elu(z0 + b0[None, :])
    zb1 = jax.nn.elu(z1 + b1[None, :])
    q0 = (jnp.tanh(zb0 @ Ws1 + bs1) @ Ws2).mean(0)
    q1 = (jnp.tanh(zb1 @ Ws1 + bs1) @ Ws2).mean(0)
    beta = jax.nn.softmax(jnp.stack([q0, q1, ]).reshape(2), axis=0)
    return beta[0] * zb0 + beta[1] * zb1


# speedup vs baseline: 14.1355x; 13.7066x over previous
import functools

import jax
import jax.numpy as jnp
from jax import lax
from jax.experimental import pallas as pl
from jax.experimental.pallas import tpu as pltpu
from jax.experimental.pallas import tpu_sc as plsc

N = 10000
NP = 10240
IN = 128
H = 8
D = 64
HD = 512
E = 320000
HID = 128

NT = 16
NC = 2
TILE_E = E // NT
C = 80
NCH = TILE_E // C
RPT = NP // NT


def _dense_body(h_ref, w_ref, o_ref):
    o_ref[0] = jnp.dot(h_ref[...], w_ref[...], preferred_element_type=jnp.float32)


def _dense(hp, wext):
    nslice = wext.shape[1] // 128
    return pl.pallas_call(
        _dense_body,
        grid=(NP // 256, nslice),
        in_specs=[
            pl.BlockSpec((256, IN), lambda i, j: (i, 0)),
            pl.BlockSpec((IN, 128), lambda i, j: (0, j)),
        ],
        out_specs=pl.BlockSpec((1, 256, 128), lambda i, j: (j, i, 0)),
        out_shape=jax.ShapeDtypeStruct((nslice, NP, 128), jnp.float32),
    )(hp, wext)


def _dyng(v, idx):
    dnums = lax.GatherDimensionNumbers(
        offset_dims=(), collapsed_slice_dims=(0,), start_index_map=(0,))
    return lax.gather(v, idx[:, None], dnums, (1,),
                      mode=lax.GatherScatterMode.PROMISE_IN_BOUNDS)


def _gat_edges_sc(src, dst, elr, ftab):
    mesh = plsc.VectorSubcoreMesh(core_axis_name="c", subcore_axis_name="s")

    @functools.partial(
        pl.kernel,
        out_type=jax.ShapeDtypeStruct((4, NP, 128), jnp.float32),
        mesh=mesh,
        compiler_params=pltpu.CompilerParams(use_tc_tiling_on_sc=False),
        scratch_types=[
            pltpu.VMEM((C,), jnp.int32),
            pltpu.VMEM((C,), jnp.int32),
            pltpu.VMEM((C,), jnp.int32),
            pltpu.VMEM((C, 16), jnp.float32),
            pltpu.VMEM((C, 16), jnp.float32),
            pltpu.VMEM((C, 16), jnp.float32),
            pltpu.VMEM((C, 16), jnp.float32),
            pltpu.VMEM((C, 128), jnp.float32),
            pltpu.VMEM_SHARED((NP, 16), jnp.float32),
            pltpu.VMEM_SHARED((NP, 16), jnp.float32),
            pltpu.VMEM_SHARED((NP, 128), jnp.float32),
        ],
    )
    def k(src_h, dst_h, elr_h, ftab_h, z_h,
          sidx, didx, fidx, es, ed, sb, wb, fb, s_acc, elr_s, o_acc):
        core = lax.axis_index("c")
        t = lax.axis_index("s")
        base = t * TILE_E
        iota16 = lax.iota(jnp.int32, 16)

        zero16 = jnp.zeros((16,), jnp.float32)

        pltpu.sync_copy(elr_h.at[pl.ds(t * RPT, RPT)],
                        elr_s.at[pl.ds(t * RPT, RPT)])

        def zwb(i, _):
            wb[i] = zero16
            return 0
        lax.fori_loop(0, C, zwb, 0)
        for j in range(RPT // C):
            pltpu.sync_copy(wb, s_acc.at[pl.ds(t * RPT + j * C, C)])
        plsc.subcore_barrier()

        col8 = jnp.bitwise_and(iota16 + 8, 15)

        def edge_stats(i, _):
            va = es[i]
            vb = _dyng(ed[i], col8)
            e = va + vb
            w = jnp.exp(jnp.maximum(e, 0.2 * e))
            wb[i] = w
            return 0

        def chunk_a(kk, _):
            off = base + kk * C
            pltpu.sync_copy(src_h.at[pl.ds(off, C)], sidx)
            pltpu.sync_copy(dst_h.at[pl.ds(off, C)], didx)
            pltpu.sync_copy(elr_s.at[sidx], es)
            pltpu.sync_copy(elr_s.at[didx], ed)
            lax.fori_loop(0, C, edge_stats, 0)
            pltpu.sync_copy(wb, s_acc.at[didx], add=True)
            return 0
        lax.fori_loop(0, NCH, chunk_a, 0)
        plsc.subcore_barrier()

        for rr in range(2):
            r = 2 * core + rr
            def zfb(i, _):
                for j in range(8):
                    fb[i, pl.ds(j * 16, 16)] = zero16
                return 0
            lax.fori_loop(0, C, zfb, 0)
            for j in range(RPT // C):
                pltpu.sync_copy(fb, o_acc.at[pl.ds(t * RPT + j * C, C)])
            plsc.subcore_barrier()

            def edge_apply(i, _):
                va = es[i]
                vb = _dyng(ed[i], col8)
                e = va + vb
                w = jnp.exp(jnp.maximum(e, 0.2 * e))
                sv = sb[i]
                alpha = w / (sv + 1e-9)
                a0 = _dyng(alpha, jnp.full((16,), 2 * r, jnp.int32))
                a1 = _dyng(alpha, jnp.full((16,), 2 * r + 1, jnp.int32))
                for j in range(8):
                    a = a0 if j < 4 else a1
                    fb[i, pl.ds(j * 16, 16)] = fb[i, pl.ds(j * 16, 16)] * a
                return 0

            def chunk_b(kk, _):
                off = base + kk * C
                pltpu.sync_copy(src_h.at[pl.ds(off, C)], sidx)
                pltpu.sync_copy(dst_h.at[pl.ds(off, C)], didx)
                pltpu.sync_copy(elr_s.at[sidx], es)
                pltpu.sync_copy(elr_s.at[didx], ed)
                pltpu.sync_copy(s_acc.at[didx], sb)
                def fidx_set(j, _):
                    fidx[pl.ds(j * 16, 16)] = sidx[pl.ds(j * 16, 16)] + r * NP
                    return 0
                lax.fori_loop(0, C // 16, fidx_set, 0)
                pltpu.sync_copy(ftab_h.at[fidx], fb)
                lax.fori_loop(0, C, edge_apply, 0)
                pltpu.sync_copy(fb, o_acc.at[didx], add=True)
                return 0
            lax.fori_loop(0, NCH, chunk_b, 0)
            plsc.subcore_barrier()

            pltpu.sync_copy(o_acc.at[pl.ds(t * RPT, RPT)],
                            z_h.at[r, pl.ds(t * RPT, RPT)])
            plsc.subcore_barrier()

    return k(src, dst, elr, ftab)


def _elu(x):
    return jnp.where(x > 0, x, jnp.exp(jnp.minimum(x, 0.0)) - 1.0)


def _k3a_body(z0_ref, z1_ref, b0_ref, b1_ref, ws1_ref, bs1_ref, ws2_ref, o_ref):
    i = pl.program_id(0)
    acc0 = jnp.zeros((256, HID), jnp.float32)
    acc1 = jnp.zeros((256, HID), jnp.float32)
    for r in range(4):
        zb0 = _elu(z0_ref[r] + b0_ref[r][None, :])
        zb1 = _elu(z1_ref[r] + b1_ref[r][None, :])
        acc0 += jnp.dot(zb0, ws1_ref[r], preferred_element_type=jnp.float32)
        acc1 += jnp.dot(zb1, ws1_ref[r], preferred_element_type=jnp.float32)
    t0 = jnp.tanh(acc0 + bs1_ref[0][None, :])
    t1 = jnp.tanh(acc1 + bs1_ref[0][None, :])
    q0 = jnp.sum(t0 * ws2_ref[0][None, :], axis=1)
    q1 = jnp.sum(t1 * ws2_ref[0][None, :], axis=1)
    rowid = i * 256 + lax.iota(jnp.int32, 256)
    valid = (rowid < N).astype(jnp.float32)
    s0 = jnp.sum(q0 * valid)
    s1 = jnp.sum(q1 * valid)
    col = lax.broadcasted_iota(jnp.int32, (1, 1, 128), 2)
    o_ref[...] = jnp.where(col == 0, s0, jnp.where(col == 1, s1, 0.0))


def _k3b_body(z0_ref, z1_ref, b0_ref, b1_ref, beta_ref, o_ref):
    be0 = beta_ref[0, 0]
    be1 = beta_ref[1, 0]
    for r in range(4):
        zb0 = _elu(z0_ref[r] + b0_ref[r][None, :])
        zb1 = _elu(z1_ref[r] + b1_ref[r][None, :])
        o_ref[:, pl.ds(r * 128, 128)] = be0 * zb0 + be1 * zb1


def kernel(h, edge_index0, edge_index1, W0, al0, ar0, b0, W1, al1, ar1, b1, Ws1, bs1, Ws2):
    h = h.astype(jnp.float32)
    hp = jnp.pad(h, ((0, NP - N), (0, 0)))
    wcat = jnp.concatenate([W0, W1], axis=1)
    eye = jnp.eye(H, dtype=jnp.float32)
    def blk(a):
        return (a[:, None, :] * eye[:, :, None]).transpose(1, 2, 0).reshape(HD, H)
    z96 = jnp.zeros((HD, 128 - 4 * H), jnp.float32)
    zH2 = jnp.zeros((HD, 2 * H), jnp.float32)
    B0 = jnp.concatenate([blk(al0), blk(ar0), zH2, z96], axis=1)
    B1 = jnp.concatenate([zH2, blk(al1), blk(ar1), z96], axis=1)
    Bm = jnp.concatenate([B0, B1], axis=0)
    WB = wcat @ Bm
    wext = jnp.concatenate([wcat, WB], axis=1)

    fout = _dense(hp, wext)
    EL = fout[8]
    elr0 = EL[:, 0:16]
    elr1 = EL[:, 16:32]
    ftab0 = fout[0:4].reshape(4 * NP, 128)
    ftab1 = fout[4:8].reshape(4 * NP, 128)

    src0 = edge_index0[0].astype(jnp.int32)
    dst0 = edge_index0[1].astype(jnp.int32)
    src1 = edge_index1[0].astype(jnp.int32)
    dst1 = edge_index1[1].astype(jnp.int32)

    z0 = _gat_edges_sc(src0, dst0, elr0, ftab0)
    z1 = _gat_edges_sc(src1, dst1, elr1, ftab1)

    b0m = b0.reshape(4, 128)
    b1m = b1.reshape(4, 128)
    ws1m = Ws1.reshape(4, 128, HID)
    bs1m = bs1.reshape(1, HID)
    ws2m = Ws2.reshape(1, HID)

    qs = pl.pallas_call(
        _k3a_body,
        grid=(NP // 256,),
        in_specs=[
            pl.BlockSpec((4, 256, 128), lambda i: (0, i, 0)),
            pl.BlockSpec((4, 256, 128), lambda i: (0, i, 0)),
            pl.BlockSpec((4, 128), lambda i: (0, 0)),
            pl.BlockSpec((4, 128), lambda i: (0, 0)),
            pl.BlockSpec((4, 128, HID), lambda i: (0, 0, 0)),
            pl.BlockSpec((1, HID), lambda i: (0, 0)),
            pl.BlockSpec((1, HID), lambda i: (0, 0)),
        ],
        out_specs=pl.BlockSpec((1, 1, 128), lambda i: (i, 0, 0)),
        out_shape=jax.ShapeDtypeStruct((NP // 256, 1, 128), jnp.float32),
    )(z0, z1, b0m, b1m, ws1m, bs1m, ws2m)

    w2 = jnp.sum(qs[:, 0, 0:2], axis=0) / N
    beta = jax.nn.softmax(w2).reshape(2, 1)

    outp = pl.pallas_call(
        _k3b_body,
        grid=(NP // 256,),
        in_specs=[
            pl.BlockSpec((4, 256, 128), lambda i: (0, i, 0)),
            pl.BlockSpec((4, 256, 128), lambda i: (0, i, 0)),
            pl.BlockSpec((4, 128), lambda i: (0, 0)),
            pl.BlockSpec((4, 128), lambda i: (0, 0)),
            pl.BlockSpec((2, 1), lambda i: (0, 0), memory_space=pltpu.SMEM),
        ],
        out_specs=pl.BlockSpec((256, HD), lambda i: (i, 0)),
        out_shape=jax.ShapeDtypeStruct((NP, HD), jnp.float32),
    )(z0, z1, b0m, b1m, beta)

    return outp[:N]


# speedup vs baseline: 17.1886x; 1.2160x over previous
import functools

import jax
import jax.numpy as jnp
from jax import lax
from jax.experimental import pallas as pl
from jax.experimental.pallas import tpu as pltpu
from jax.experimental.pallas import tpu_sc as plsc

N = 10000
NP = 10240
IN = 128
H = 8
D = 64
HD = 512
E = 320000
HID = 128

NT = 16
NC = 2
TILE_E = E // NT
C = 80
NCH = TILE_E // C
RPT = NP // NT


def _dense_body(h_ref, w_ref, o_ref):
    o_ref[0] = jnp.dot(h_ref[...], w_ref[...], preferred_element_type=jnp.float32)


def _dense(hp, wext):
    nslice = wext.shape[1] // 128
    return pl.pallas_call(
        _dense_body,
        grid=(NP // 256, nslice),
        in_specs=[
            pl.BlockSpec((256, IN), lambda i, j: (i, 0)),
            pl.BlockSpec((IN, 128), lambda i, j: (0, j)),
        ],
        out_specs=pl.BlockSpec((1, 256, 128), lambda i, j: (j, i, 0)),
        out_shape=jax.ShapeDtypeStruct((nslice, NP, 128), jnp.float32),
    )(hp, wext)


def _dyng(v, idx):
    dnums = lax.GatherDimensionNumbers(
        offset_dims=(), collapsed_slice_dims=(0,), start_index_map=(0,))
    return lax.gather(v, idx[:, None], dnums, (1,),
                      mode=lax.GatherScatterMode.PROMISE_IN_BOUNDS)


def _gat_edges_sc(src, dst, elr, ftab):
    mesh = plsc.VectorSubcoreMesh(core_axis_name="c", subcore_axis_name="s")

    @functools.partial(
        pl.kernel,
        out_type=jax.ShapeDtypeStruct((4, NP, 128), jnp.float32),
        mesh=mesh,
        compiler_params=pltpu.CompilerParams(use_tc_tiling_on_sc=False),
        scratch_types=[
            [pltpu.VMEM((C,), jnp.int32) for _ in range(2)],
            [pltpu.VMEM((C,), jnp.int32) for _ in range(2)],
            [pltpu.VMEM((C,), jnp.int32) for _ in range(2)],
            pltpu.VMEM((C, 16), jnp.float32),
            pltpu.VMEM((C, 16), jnp.float32),
            pltpu.VMEM((C, 16), jnp.float32),
            pltpu.VMEM((C, 16), jnp.float32),
            [pltpu.VMEM((C, 128), jnp.float32) for _ in range(2)],
            [pltpu.SemaphoreType.DMA for _ in range(2)],
            pltpu.VMEM_SHARED((NP, 16), jnp.float32),
            pltpu.VMEM_SHARED((NP, 16), jnp.float32),
            pltpu.VMEM_SHARED((NP, 128), jnp.float32),
        ],
    )
    def k(src_h, dst_h, elr_h, ftab_h, z_h,
          sidx, didx, fidx, es, ed, sb, wb, fb, fsem, s_acc, elr_s, o_acc):
        core = lax.axis_index("c")
        t = lax.axis_index("s")
        base = t * TILE_E
        iota16 = lax.iota(jnp.int32, 16)

        zero16 = jnp.zeros((16,), jnp.float32)

        pltpu.sync_copy(elr_h.at[pl.ds(t * RPT, RPT)],
                        elr_s.at[pl.ds(t * RPT, RPT)])

        def zwb(i, _):
            wb[i] = zero16
            return 0
        lax.fori_loop(0, C, zwb, 0)
        for j in range(RPT // C):
            pltpu.sync_copy(wb, s_acc.at[pl.ds(t * RPT + j * C, C)])
        plsc.subcore_barrier()

        col8 = jnp.bitwise_and(iota16 + 8, 15)

        def edge_stats(i, _):
            va = es[i]
            vb = _dyng(ed[i], col8)
            e = va + vb
            w = jnp.exp(jnp.maximum(e, 0.2 * e))
            wb[i] = w
            return 0

        def chunk_a(kk, _):
            off = base + kk * C
            pltpu.sync_copy(src_h.at[pl.ds(off, C)], sidx[0])
            pltpu.sync_copy(dst_h.at[pl.ds(off, C)], didx[0])
            pltpu.sync_copy(elr_s.at[sidx[0]], es)
            pltpu.sync_copy(elr_s.at[didx[0]], ed)
            lax.fori_loop(0, C, edge_stats, 0)
            pltpu.sync_copy(wb, s_acc.at[didx[0]], add=True)
            return 0
        lax.fori_loop(0, NCH, chunk_a, 0)
        plsc.subcore_barrier()

        for rr in range(2):
            r = 2 * core + rr
            def zfb(i, _):
                for j in range(8):
                    fb[0][i, pl.ds(j * 16, 16)] = zero16
                return 0
            lax.fori_loop(0, C, zfb, 0)
            for j in range(RPT // C):
                pltpu.sync_copy(fb[0], o_acc.at[pl.ds(t * RPT + j * C, C)])
            plsc.subcore_barrier()

            def mk_edge_apply(b):
                def edge_apply(i, _):
                    va = es[i]
                    vb = _dyng(ed[i], col8)
                    e = va + vb
                    w = jnp.exp(jnp.maximum(e, 0.2 * e))
                    sv = sb[i]
                    alpha = w / (sv + 1e-9)
                    a0 = _dyng(alpha, jnp.full((16,), 2 * r, jnp.int32))
                    a1 = _dyng(alpha, jnp.full((16,), 2 * r + 1, jnp.int32))
                    for j in range(8):
                        a = a0 if j < 4 else a1
                        fb[b][i, pl.ds(j * 16, 16)] = fb[b][i, pl.ds(j * 16, 16)] * a
                    return 0
                return edge_apply

            def b_prefetch(kk, b):
                off = base + kk * C
                pltpu.sync_copy(src_h.at[pl.ds(off, C)], sidx[b])
                pltpu.sync_copy(dst_h.at[pl.ds(off, C)], didx[b])
                def fidx_set(j, _):
                    fidx[b][pl.ds(j * 16, 16)] = sidx[b][pl.ds(j * 16, 16)] + r * NP
                    return 0
                lax.fori_loop(0, C // 16, fidx_set, 0)
                pltpu.async_copy(ftab_h.at[fidx[b]], fb[b], fsem[b])

            def b_finish(b):
                pltpu.sync_copy(elr_s.at[sidx[b]], es)
                pltpu.sync_copy(elr_s.at[didx[b]], ed)
                pltpu.sync_copy(s_acc.at[didx[b]], sb)
                pltpu.make_async_copy(ftab_h.at[fidx[b]], fb[b], fsem[b]).wait()
                lax.fori_loop(0, C, mk_edge_apply(b), 0)
                pltpu.sync_copy(fb[b], o_acc.at[didx[b]], add=True)

            b_prefetch(0, 0)

            def b_pair(q, _):
                b_prefetch(2 * q + 1, 1)
                b_finish(0)

                @pl.when(2 * q + 2 < NCH)
                def _():
                    b_prefetch(2 * q + 2, 0)
                b_finish(1)
                return 0
            lax.fori_loop(0, NCH // 2, b_pair, 0)
            plsc.subcore_barrier()

            pltpu.sync_copy(o_acc.at[pl.ds(t * RPT, RPT)],
                            z_h.at[r, pl.ds(t * RPT, RPT)])
            plsc.subcore_barrier()

    return k(src, dst, elr, ftab)


def _elu(x):
    return jnp.where(x > 0, x, jnp.exp(jnp.minimum(x, 0.0)) - 1.0)


def _k3a_body(z0_ref, z1_ref, b0_ref, b1_ref, ws1_ref, bs1_ref, ws2_ref, o_ref):
    i = pl.program_id(0)
    acc0 = jnp.zeros((256, HID), jnp.float32)
    acc1 = jnp.zeros((256, HID), jnp.float32)
    for r in range(4):
        zb0 = _elu(z0_ref[r] + b0_ref[r][None, :])
        zb1 = _elu(z1_ref[r] + b1_ref[r][None, :])
        acc0 += jnp.dot(zb0, ws1_ref[r], preferred_element_type=jnp.float32)
        acc1 += jnp.dot(zb1, ws1_ref[r], preferred_element_type=jnp.float32)
    t0 = jnp.tanh(acc0 + bs1_ref[0][None, :])
    t1 = jnp.tanh(acc1 + bs1_ref[0][None, :])
    q0 = jnp.sum(t0 * ws2_ref[0][None, :], axis=1)
    q1 = jnp.sum(t1 * ws2_ref[0][None, :], axis=1)
    rowid = i * 256 + lax.iota(jnp.int32, 256)
    valid = (rowid < N).astype(jnp.float32)
    s0 = jnp.sum(q0 * valid)
    s1 = jnp.sum(q1 * valid)
    col = lax.broadcasted_iota(jnp.int32, (1, 1, 128), 2)
    o_ref[...] = jnp.where(col == 0, s0, jnp.where(col == 1, s1, 0.0))


def _k3b_body(z0_ref, z1_ref, b0_ref, b1_ref, beta_ref, o_ref):
    be0 = beta_ref[0, 0]
    be1 = beta_ref[1, 0]
    for r in range(4):
        zb0 = _elu(z0_ref[r] + b0_ref[r][None, :])
        zb1 = _elu(z1_ref[r] + b1_ref[r][None, :])
        o_ref[:, pl.ds(r * 128, 128)] = be0 * zb0 + be1 * zb1


def kernel(h, edge_index0, edge_index1, W0, al0, ar0, b0, W1, al1, ar1, b1, Ws1, bs1, Ws2):
    h = h.astype(jnp.float32)
    hp = jnp.pad(h, ((0, NP - N), (0, 0)))
    wcat = jnp.concatenate([W0, W1], axis=1)
    eye = jnp.eye(H, dtype=jnp.float32)
    def blk(a):
        return (a[:, None, :] * eye[:, :, None]).transpose(1, 2, 0).reshape(HD, H)
    z96 = jnp.zeros((HD, 128 - 4 * H), jnp.float32)
    zH2 = jnp.zeros((HD, 2 * H), jnp.float32)
    B0 = jnp.concatenate([blk(al0), blk(ar0), zH2, z96], axis=1)
    B1 = jnp.concatenate([zH2, blk(al1), blk(ar1), z96], axis=1)
    Bm = jnp.concatenate([B0, B1], axis=0)
    WB = wcat @ Bm
    wext = jnp.concatenate([wcat, WB], axis=1)

    fout = _dense(hp, wext)
    EL = fout[8]
    elr0 = EL[:, 0:16]
    elr1 = EL[:, 16:32]
    ftab0 = fout[0:4].reshape(4 * NP, 128)
    ftab1 = fout[4:8].reshape(4 * NP, 128)

    src0 = edge_index0[0].astype(jnp.int32)
    dst0 = edge_index0[1].astype(jnp.int32)
    src1 = edge_index1[0].astype(jnp.int32)
    dst1 = edge_index1[1].astype(jnp.int32)

    z0 = _gat_edges_sc(src0, dst0, elr0, ftab0)
    z1 = _gat_edges_sc(src1, dst1, elr1, ftab1)

    b0m = b0.reshape(4, 128)
    b1m = b1.reshape(4, 128)
    ws1m = Ws1.reshape(4, 128, HID)
    bs1m = bs1.reshape(1, HID)
    ws2m = Ws2.reshape(1, HID)

    qs = pl.pallas_call(
        _k3a_body,
        grid=(NP // 256,),
        in_specs=[
            pl.BlockSpec((4, 256, 128), lambda i: (0, i, 0)),
            pl.BlockSpec((4, 256, 128), lambda i: (0, i, 0)),
            pl.BlockSpec((4, 128), lambda i: (0, 0)),
            pl.BlockSpec((4, 128), lambda i: (0, 0)),
            pl.BlockSpec((4, 128, HID), lambda i: (0, 0, 0)),
            pl.BlockSpec((1, HID), lambda i: (0, 0)),
            pl.BlockSpec((1, HID), lambda i: (0, 0)),
        ],
        out_specs=pl.BlockSpec((1, 1, 128), lambda i: (i, 0, 0)),
        out_shape=jax.ShapeDtypeStruct((NP // 256, 1, 128), jnp.float32),
    )(z0, z1, b0m, b1m, ws1m, bs1m, ws2m)

    w2 = jnp.sum(qs[:, 0, 0:2], axis=0) / N
    beta = jax.nn.softmax(w2).reshape(2, 1)

    outp = pl.pallas_call(
        _k3b_body,
        grid=(NP // 256,),
        in_specs=[
            pl.BlockSpec((4, 256, 128), lambda i: (0, i, 0)),
            pl.BlockSpec((4, 256, 128), lambda i: (0, i, 0)),
            pl.BlockSpec((4, 128), lambda i: (0, 0)),
            pl.BlockSpec((4, 128), lambda i: (0, 0)),
            pl.BlockSpec((2, 1), lambda i: (0, 0), memory_space=pltpu.SMEM),
        ],
        out_specs=pl.BlockSpec((256, HD), lambda i: (i, 0)),
        out_shape=jax.ShapeDtypeStruct((NP, HD), jnp.float32),
    )(z0, z1, b0m, b1m, beta)

    return outp[:N]


# speedup vs baseline: 19.5594x; 1.1379x over previous
import functools

import jax
import jax.numpy as jnp
from jax import lax
from jax.experimental import pallas as pl
from jax.experimental.pallas import tpu as pltpu
from jax.experimental.pallas import tpu_sc as plsc

N = 10000
NP = 10240
IN = 128
H = 8
D = 64
HD = 512
E = 320000
HID = 128

NT = 16
NC = 2
TILE_E = E // NT
C = 80
NCH = TILE_E // C
RPT = NP // NT


def _dense_body(h_ref, w_ref, o_ref):
    o_ref[0] = jnp.dot(h_ref[...], w_ref[...], preferred_element_type=jnp.float32)


def _dense(hp, wext):
    nslice = wext.shape[1] // 128
    return pl.pallas_call(
        _dense_body,
        grid=(NP // 256, nslice),
        in_specs=[
            pl.BlockSpec((256, IN), lambda i, j: (i, 0)),
            pl.BlockSpec((IN, 128), lambda i, j: (0, j)),
        ],
        out_specs=pl.BlockSpec((1, 256, 128), lambda i, j: (j, i, 0)),
        out_shape=jax.ShapeDtypeStruct((nslice, NP, 128), jnp.float32),
    )(hp, wext)


def _dyng(v, idx):
    dnums = lax.GatherDimensionNumbers(
        offset_dims=(), collapsed_slice_dims=(0,), start_index_map=(0,))
    return lax.gather(v, idx[:, None], dnums, (1,),
                      mode=lax.GatherScatterMode.PROMISE_IN_BOUNDS)


def _gat_edges_sc(src, dst, elr, ftab):
    mesh = plsc.VectorSubcoreMesh(core_axis_name="c", subcore_axis_name="s")

    @functools.partial(
        pl.kernel,
        out_type=jax.ShapeDtypeStruct((4, N, 128), jnp.float32),
        mesh=mesh,
        compiler_params=pltpu.CompilerParams(use_tc_tiling_on_sc=False),
        scratch_types=[
            [pltpu.VMEM((C,), jnp.int32) for _ in range(2)],
            [pltpu.VMEM((C,), jnp.int32) for _ in range(2)],
            [pltpu.VMEM((C,), jnp.int32) for _ in range(2)],
            [pltpu.VMEM((C, 16), jnp.float32) for _ in range(2)],
            [pltpu.VMEM((C, 16), jnp.float32) for _ in range(2)],
            [pltpu.VMEM((C, 16), jnp.float32) for _ in range(2)],
            pltpu.VMEM((C, 16), jnp.float32),
            [pltpu.VMEM((C, 128), jnp.float32) for _ in range(2)],
            [pltpu.SemaphoreType.DMA for _ in range(2)],
            [pltpu.SemaphoreType.DMA for _ in range(2)],
            pltpu.VMEM_SHARED((N, 16), jnp.float32),
            pltpu.VMEM_SHARED((N, 16), jnp.float32),
            pltpu.VMEM_SHARED((N, 128), jnp.float32),
        ],
    )
    def k(src_h, dst_h, elr_h, ftab_h, z_h,
          sidx, didx, fidx, es, ed, sb, wb, fb, fsem, asem,
          s_acc, elr_s, o_acc):
        core = lax.axis_index("c")
        t = lax.axis_index("s")
        base = t * TILE_E
        iota16 = lax.iota(jnp.int32, 16)

        zero16 = jnp.zeros((16,), jnp.float32)

        srpt = N // NT
        pltpu.sync_copy(elr_h.at[pl.ds(t * srpt, srpt)],
                        elr_s.at[pl.ds(t * srpt, srpt)])

        def zwb(i, _):
            wb[i] = zero16
            return 0
        lax.fori_loop(0, C, zwb, 0)

        def zs(j, _):
            pltpu.sync_copy(wb.at[pl.ds(0, 25)],
                            s_acc.at[pl.ds(t * srpt + j * 25, 25)])
            return 0
        lax.fori_loop(0, srpt // 25, zs, 0)
        plsc.subcore_barrier()

        col8 = jnp.bitwise_and(iota16 + 8, 15)

        def mk_edge_stats(b):
            def edge_stats(i, _):
                va = es[b][i]
                vb = _dyng(ed[b][i], col8)
                e = va + vb
                w = jnp.exp(jnp.maximum(e, 0.2 * e))
                wb[i] = w
                return 0
            return edge_stats

        def a_prefetch(kk, b):
            off = base + kk * C
            pltpu.sync_copy(src_h.at[pl.ds(off, C)], sidx[b])
            pltpu.sync_copy(dst_h.at[pl.ds(off, C)], didx[b])
            pltpu.async_copy(elr_s.at[sidx[b]], es[b], asem[b])
            pltpu.async_copy(elr_s.at[didx[b]], ed[b], asem[b])

        def a_finish(b):
            pltpu.make_async_copy(elr_s.at[sidx[b]], es[b], asem[b]).wait()
            pltpu.make_async_copy(elr_s.at[didx[b]], ed[b], asem[b]).wait()
            lax.fori_loop(0, C, mk_edge_stats(b), 0)
            pltpu.sync_copy(wb, s_acc.at[didx[b]], add=True)

        a_prefetch(0, 0)

        def a_pair(q, _):
            a_prefetch(2 * q + 1, 1)
            a_finish(0)

            @pl.when(2 * q + 2 < NCH)
            def _():
                a_prefetch(2 * q + 2, 0)
            a_finish(1)
            return 0
        lax.fori_loop(0, NCH // 2, a_pair, 0)
        plsc.subcore_barrier()

        for rr in range(2):
            r = 2 * core + rr
            def zfb(i, _):
                for j in range(8):
                    fb[0][i, pl.ds(j * 16, 16)] = zero16
                return 0
            lax.fori_loop(0, C, zfb, 0)

            def zo(j, _):
                pltpu.sync_copy(fb[0].at[pl.ds(0, 25)],
                                o_acc.at[pl.ds(t * srpt + j * 25, 25)])
                return 0
            lax.fori_loop(0, srpt // 25, zo, 0)
            plsc.subcore_barrier()

            def mk_edge_apply(b):
                def edge_apply(i, _):
                    va = es[b][i]
                    vb = _dyng(ed[b][i], col8)
                    e = va + vb
                    w = jnp.exp(jnp.maximum(e, 0.2 * e))
                    sv = sb[b][i]
                    alpha = w / (sv + 1e-9)
                    a0 = _dyng(alpha, jnp.full((16,), 2 * r, jnp.int32))
                    a1 = _dyng(alpha, jnp.full((16,), 2 * r + 1, jnp.int32))
                    for j in range(8):
                        a = a0 if j < 4 else a1
                        fb[b][i, pl.ds(j * 16, 16)] = fb[b][i, pl.ds(j * 16, 16)] * a
                    return 0
                return edge_apply

            def b_prefetch(kk, b):
                off = base + kk * C
                pltpu.sync_copy(src_h.at[pl.ds(off, C)], sidx[b])
                pltpu.sync_copy(dst_h.at[pl.ds(off, C)], didx[b])
                def fidx_set(j, _):
                    fidx[b][pl.ds(j * 16, 16)] = sidx[b][pl.ds(j * 16, 16)] + r * NP
                    return 0
                lax.fori_loop(0, C // 16, fidx_set, 0)
                pltpu.async_copy(ftab_h.at[fidx[b]], fb[b], fsem[b])
                pltpu.async_copy(elr_s.at[sidx[b]], es[b], asem[b])
                pltpu.async_copy(elr_s.at[didx[b]], ed[b], asem[b])
                pltpu.async_copy(s_acc.at[didx[b]], sb[b], asem[b])

            def b_finish(b):
                pltpu.make_async_copy(elr_s.at[sidx[b]], es[b], asem[b]).wait()
                pltpu.make_async_copy(elr_s.at[didx[b]], ed[b], asem[b]).wait()
                pltpu.make_async_copy(s_acc.at[didx[b]], sb[b], asem[b]).wait()
                pltpu.make_async_copy(ftab_h.at[fidx[b]], fb[b], fsem[b]).wait()
                lax.fori_loop(0, C, mk_edge_apply(b), 0)
                pltpu.sync_copy(fb[b], o_acc.at[didx[b]], add=True)

            b_prefetch(0, 0)

            def b_pair(q, _):
                b_prefetch(2 * q + 1, 1)
                b_finish(0)

                @pl.when(2 * q + 2 < NCH)
                def _():
                    b_prefetch(2 * q + 2, 0)
                b_finish(1)
                return 0
            lax.fori_loop(0, NCH // 2, b_pair, 0)
            plsc.subcore_barrier()

            pltpu.sync_copy(o_acc.at[pl.ds(t * srpt, srpt)],
                            z_h.at[r, pl.ds(t * srpt, srpt)])
            plsc.subcore_barrier()

    return k(src, dst, elr, ftab)


def _elu(x):
    return jnp.where(x > 0, x, jnp.exp(jnp.minimum(x, 0.0)) - 1.0)


def _k3a_body(z0_ref, z1_ref, b0_ref, b1_ref, ws1_ref, bs1_ref, ws2_ref, o_ref):
    i = pl.program_id(0)
    acc0 = jnp.zeros((256, HID), jnp.float32)
    acc1 = jnp.zeros((256, HID), jnp.float32)
    for r in range(4):
        zb0 = _elu(z0_ref[r] + b0_ref[r][None, :])
        zb1 = _elu(z1_ref[r] + b1_ref[r][None, :])
        acc0 += jnp.dot(zb0, ws1_ref[r], preferred_element_type=jnp.float32)
        acc1 += jnp.dot(zb1, ws1_ref[r], preferred_element_type=jnp.float32)
    t0 = jnp.tanh(acc0 + bs1_ref[0][None, :])
    t1 = jnp.tanh(acc1 + bs1_ref[0][None, :])
    q0 = jnp.sum(t0 * ws2_ref[0][None, :], axis=1)
    q1 = jnp.sum(t1 * ws2_ref[0][None, :], axis=1)
    rowid = i * 256 + lax.iota(jnp.int32, 256)
    valid = (rowid < N).astype(jnp.float32)
    s0 = jnp.sum(q0 * valid)
    s1 = jnp.sum(q1 * valid)
    col = lax.broadcasted_iota(jnp.int32, (1, 1, 128), 2)
    o_ref[...] = jnp.where(col == 0, s0, jnp.where(col == 1, s1, 0.0))


def _k3b_body(z0_ref, z1_ref, b0_ref, b1_ref, beta_ref, o_ref):
    be0 = beta_ref[0, 0]
    be1 = beta_ref[1, 0]
    for r in range(4):
        zb0 = _elu(z0_ref[r] + b0_ref[r][None, :])
        zb1 = _elu(z1_ref[r] + b1_ref[r][None, :])
        o_ref[:, pl.ds(r * 128, 128)] = be0 * zb0 + be1 * zb1


def kernel(h, edge_index0, edge_index1, W0, al0, ar0, b0, W1, al1, ar1, b1, Ws1, bs1, Ws2):
    h = h.astype(jnp.float32)
    hp = jnp.pad(h, ((0, NP - N), (0, 0)))
    wcat = jnp.concatenate([W0, W1], axis=1)
    eye = jnp.eye(H, dtype=jnp.float32)
    def blk(a):
        return (a[:, None, :] * eye[:, :, None]).transpose(1, 2, 0).reshape(HD, H)
    z96 = jnp.zeros((HD, 128 - 4 * H), jnp.float32)
    zH2 = jnp.zeros((HD, 2 * H), jnp.float32)
    B0 = jnp.concatenate([blk(al0), blk(ar0), zH2, z96], axis=1)
    B1 = jnp.concatenate([zH2, blk(al1), blk(ar1), z96], axis=1)
    Bm = jnp.concatenate([B0, B1], axis=0)
    WB = wcat @ Bm
    wext = jnp.concatenate([wcat, WB], axis=1)

    fout = _dense(hp, wext)
    EL = fout[8]
    elr0 = EL[:, 0:16]
    elr1 = EL[:, 16:32]
    ftab0 = fout[0:4].reshape(4 * NP, 128)
    ftab1 = fout[4:8].reshape(4 * NP, 128)

    src0 = edge_index0[0].astype(jnp.int32)
    dst0 = edge_index0[1].astype(jnp.int32)
    src1 = edge_index1[0].astype(jnp.int32)
    dst1 = edge_index1[1].astype(jnp.int32)

    z0 = _gat_edges_sc(src0, dst0, elr0, ftab0)
    z1 = _gat_edges_sc(src1, dst1, elr1, ftab1)
    z0 = jnp.pad(z0, ((0, 0), (0, NP - N), (0, 0)))
    z1 = jnp.pad(z1, ((0, 0), (0, NP - N), (0, 0)))

    b0m = b0.reshape(4, 128)
    b1m = b1.reshape(4, 128)
    ws1m = Ws1.reshape(4, 128, HID)
    bs1m = bs1.reshape(1, HID)
    ws2m = Ws2.reshape(1, HID)

    qs = pl.pallas_call(
        _k3a_body,
        grid=(NP // 256,),
        in_specs=[
            pl.BlockSpec((4, 256, 128), lambda i: (0, i, 0)),
            pl.BlockSpec((4, 256, 128), lambda i: (0, i, 0)),
            pl.BlockSpec((4, 128), lambda i: (0, 0)),
            pl.BlockSpec((4, 128), lambda i: (0, 0)),
            pl.BlockSpec((4, 128, HID), lambda i: (0, 0, 0)),
            pl.BlockSpec((1, HID), lambda i: (0, 0)),
            pl.BlockSpec((1, HID), lambda i: (0, 0)),
        ],
        out_specs=pl.BlockSpec((1, 1, 128), lambda i: (i, 0, 0)),
        out_shape=jax.ShapeDtypeStruct((NP // 256, 1, 128), jnp.float32),
    )(z0, z1, b0m, b1m, ws1m, bs1m, ws2m)

    w2 = jnp.sum(qs[:, 0, 0:2], axis=0) / N
    beta = jax.nn.softmax(w2).reshape(2, 1)

    outp = pl.pallas_call(
        _k3b_body,
        grid=(NP // 256,),
        in_specs=[
            pl.BlockSpec((4, 256, 128), lambda i: (0, i, 0)),
            pl.BlockSpec((4, 256, 128), lambda i: (0, i, 0)),
            pl.BlockSpec((4, 128), lambda i: (0, 0)),
            pl.BlockSpec((4, 128), lambda i: (0, 0)),
            pl.BlockSpec((2, 1), lambda i: (0, 0), memory_space=pltpu.SMEM),
        ],
        out_specs=pl.BlockSpec((256, HD), lambda i: (i, 0)),
        out_shape=jax.ShapeDtypeStruct((NP, HD), jnp.float32),
    )(z0, z1, b0m, b1m, beta)

    return outp[:N]
